# Initial kernel scaffold; baseline (speedup 1.0000x reference)
import functools

import jax
import jax.numpy as jnp
from jax import lax
from jax.experimental import pallas as pl
from jax.experimental.pallas import tpu as pltpu
from jax.experimental.pallas import tpu_sc as plsc

VOCAB = 100000
D = 1024
PROMPT_LEN = 100
ID_OFFSET = 100000
B, S = 4, 2048
N = B * S
NC, NS, L = 2, 16, 16
NW = NC * NS
TPW = N // NW
CH = 32
NCH = TPW // CH

_mesh = plsc.VectorSubcoreMesh(core_axis_name="c", subcore_axis_name="s")


@functools.partial(
    pl.kernel,
    out_type=jax.ShapeDtypeStruct((N, D), jnp.float32),
    mesh=_mesh,
    scratch_types=[
        pltpu.VMEM((TPW,), jnp.int32),
        pltpu.VMEM((CH,), jnp.int32),
        pltpu.VMEM((CH,), jnp.int32),
        pltpu.VMEM((CH, D), jnp.float32),
        pltpu.VMEM((CH, D), jnp.float32),
        pltpu.SemaphoreType.DMA,
        pltpu.SemaphoreType.DMA,
    ],
)
def _embed_lookup(ids_hbm, w_hbm, p_hbm, out_hbm,
                  ids_v, idxw_v, idxp_v, rw, rp, semw, semp):
    wid = lax.axis_index("s") * NC + lax.axis_index("c")
    base = wid * TPW
    pltpu.sync_copy(ids_hbm.at[pl.ds(base, TPW)], ids_v)

    for k in range(NCH):
        off = k * CH
        cnt = jnp.zeros((L,), jnp.int32)
        for j in range(CH // L):
            ids = ids_v[pl.ds(off + j * L, L)]
            pr = ids >= ID_OFFSET
            idxw_v[pl.ds(j * L, L)] = jnp.where(pr, 0, ids)
            idxp_v[pl.ds(j * L, L)] = jnp.where(pr, ids - ID_OFFSET, 0)
            cnt = cnt + pr.astype(jnp.int32)
        nprompt = jnp.sum(cnt)

        pltpu.async_copy(w_hbm.at[idxw_v], rw, semw).wait()

        @pl.when(nprompt > 0)
        def _fix_prompt_rows():
            pltpu.async_copy(p_hbm.at[idxp_v], rp, semp).wait()

            def row_body(r, carry):
                idv = plsc.load_gather(
                    ids_v, [jnp.full((L,), off, jnp.int32) + r])
                cond = idv >= ID_OFFSET
                for j2 in range(D // L):
                    w = rw[r, pl.ds(j2 * L, L)]
                    p = rp[r, pl.ds(j2 * L, L)]
                    rw[r, pl.ds(j2 * L, L)] = jnp.where(cond, p, w)
                return carry

            lax.fori_loop(0, CH, row_body, 0)

        pltpu.sync_copy(rw, out_hbm.at[pl.ds(base + off, CH)])


def kernel(input_ids, labels, W, P):
    del labels
    ids = input_ids.reshape(-1)
    out = _embed_lookup(ids, W, P)
    return out.reshape(B, S, D)


# speedup vs baseline: 1.4244x; 1.4244x over previous
import functools

import jax
import jax.numpy as jnp
from jax import lax
from jax.experimental import pallas as pl
from jax.experimental.pallas import tpu as pltpu
from jax.experimental.pallas import tpu_sc as plsc

VOCAB = 100000
D = 1024
PROMPT_LEN = 100
ID_OFFSET = 100000
B, S = 4, 2048
N = B * S
NC, NS, L = 2, 16, 16
NW = NC * NS
TPW = N // NW
CH = 32
NCH = TPW // CH

_mesh = plsc.VectorSubcoreMesh(core_axis_name="c", subcore_axis_name="s")


@functools.partial(
    pl.kernel,
    out_type=jax.ShapeDtypeStruct((N, D), jnp.float32),
    mesh=_mesh,
    scratch_types=[
        pltpu.VMEM((TPW,), jnp.int32),
        pltpu.VMEM((CH,), jnp.int32),
        pltpu.VMEM((CH, D), jnp.float32),
        pltpu.SemaphoreType.DMA,
    ],
)
def _embed_lookup(ids_hbm, w_hbm, p_hbm, out_hbm, ids_v, idxw_v, rw, semw):
    wid = lax.axis_index("s") * NC + lax.axis_index("c")
    base = wid * TPW
    pltpu.sync_copy(ids_hbm.at[pl.ds(base, TPW)], ids_v)

    zeros = jnp.zeros((L,), jnp.int32)
    voff = jnp.full((L,), ID_OFFSET, jnp.int32)

    for k in range(NCH):
        off = k * CH
        for j in range(CH // L):
            ids = ids_v[pl.ds(off + j * L, L)]
            pr = ids >= voff
            idxw_v[pl.ds(j * L, L)] = jnp.where(pr, zeros, ids)
        pltpu.async_copy(w_hbm.at[idxw_v], rw, semw).wait()

        for j in range(CH // L):
            grp = ids_v[pl.ds(off + j * L, L)]
            for i in range(L):
                sid = grp[i]
                row = j * L + i

                @pl.when(sid >= ID_OFFSET)
                def _patch_prompt_row():
                    pltpu.sync_copy(
                        p_hbm.at[pl.ds(sid - ID_OFFSET, 1)],
                        rw.at[pl.ds(row, 1)])

        pltpu.sync_copy(rw, out_hbm.at[pl.ds(base + off, CH)])


def kernel(input_ids, labels, W, P):
    del labels
    ids = input_ids.reshape(-1)
    out = _embed_lookup(ids, W, P)
    return out.reshape(B, S, D)


# speedup vs baseline: 1.6835x; 1.1819x over previous
import functools

import jax
import jax.numpy as jnp
from jax import lax
from jax.experimental import pallas as pl
from jax.experimental.pallas import tpu as pltpu
from jax.experimental.pallas import tpu_sc as plsc

VOCAB = 100000
D = 1024
PROMPT_LEN = 100
ID_OFFSET = 100000
B, S = 4, 2048
N = B * S
NC, NS, L = 2, 16, 16
NW = NC * NS
TPW = N // NW
CH = 32
NCH = TPW // CH

_mesh = plsc.VectorSubcoreMesh(core_axis_name="c", subcore_axis_name="s")


@functools.partial(
    pl.kernel,
    out_type=jax.ShapeDtypeStruct((N, D), jnp.float32),
    mesh=_mesh,
    scratch_types=[
        pltpu.VMEM((TPW,), jnp.int32),
        pltpu.VMEM((2, CH), jnp.int32),
        pltpu.VMEM((CH, D), jnp.float32),
        pltpu.VMEM((CH, D), jnp.float32),
        pltpu.SemaphoreType.DMA,
        pltpu.SemaphoreType.DMA,
        pltpu.SemaphoreType.DMA,
        pltpu.SemaphoreType.DMA,
    ],
)
def _embed_lookup(ids_hbm, w_hbm, p_hbm, out_hbm, ids_v, idx2_v,
                  rw0, rw1, semg0, semg1, semo0, semo1):
    wid = lax.axis_index("s") * NC + lax.axis_index("c")
    base = wid * TPW
    pltpu.sync_copy(ids_hbm.at[pl.ds(base, TPW)], ids_v)

    zeros = jnp.zeros((L,), jnp.int32)
    voff = jnp.full((L,), ID_OFFSET, jnp.int32)
    bufs = (rw0, rw1)
    semg = (semg0, semg1)
    semo = (semo0, semo1)

    def compute_idx(k, b):
        off = k * CH
        for j in range(CH // L):
            ids = ids_v[pl.ds(off + j * L, L)]
            pr = ids >= voff
            idx2_v[b, pl.ds(j * L, L)] = jnp.where(pr, zeros, ids)

    def patch_prompt_rows(k, rw):
        off = k * CH
        for j in range(CH // L):
            grp = ids_v[pl.ds(off + j * L, L)]
            for i in range(L):
                sid = grp[i]
                row = j * L + i

                @pl.when(sid >= ID_OFFSET)
                def _patch():
                    pltpu.sync_copy(
                        p_hbm.at[pl.ds(sid - ID_OFFSET, 1)],
                        rw.at[pl.ds(row, 1)])

    compute_idx(0, 0)
    gathers = [pltpu.async_copy(w_hbm.at[idx2_v.at[0]], rw0, semg0)]
    writes = []
    for k in range(NCH):
        b = k % 2
        nb = (k + 1) % 2
        if k + 1 < NCH:
            if k >= 1:
                writes[k - 1].wait()
            compute_idx(k + 1, nb)
            gathers.append(
                pltpu.async_copy(w_hbm.at[idx2_v.at[nb]], bufs[nb],
                                 semg[nb]))
        gathers[k].wait()
        patch_prompt_rows(k, bufs[b])
        writes.append(
            pltpu.async_copy(bufs[b],
                             out_hbm.at[pl.ds(base + k * CH, CH)], semo[b]))
    writes[NCH - 2].wait()
    writes[NCH - 1].wait()


def kernel(input_ids, labels, W, P):
    del labels
    ids = input_ids.reshape(-1)
    out = _embed_lookup(ids, W, P)
    return out.reshape(B, S, D)
